# Initial kernel scaffold; baseline (speedup 1.0000x reference)
import jax
import jax.numpy as jnp
from jax.experimental import pallas as pl


def kernel(x, edge_index, edge_attr, u, batch, W1, b1, g1, be1, W2, b2, g2, be2, W3, b3):
    raise NotImplementedError("write your pallas kernel here")


# speedup vs baseline: 2.2828x; 2.2828x over previous
import functools

import jax
import jax.numpy as jnp
from jax import lax
from jax.experimental import pallas as pl
from jax.experimental.pallas import tpu as pltpu
from jax.experimental.pallas import tpu_sc as plsc

F32 = jnp.float32
EPS = 1e-5
CHUNK = 128
NW = 32


def _sds(shape, dtype=F32):
    return jax.ShapeDtypeStruct(shape, dtype)


def _npad(N):
    return -(-N // (16 * CHUNK)) * (16 * CHUNK)


def _p_body(x_ref, b_ref, w_ref, wb_ref, o_ref):
    o_ref[...] = (
        jnp.dot(x_ref[...], w_ref[...], preferred_element_type=F32)
        + b_ref[...] * wb_ref[...]
    )


def _mea_body(ea_ref, m_ref):
    @pl.when(pl.program_id(0) == 0)
    def _init():
        m_ref[...] = jnp.zeros_like(m_ref)

    blk = ea_ref[...]
    m_ref[...] += lax.dot_general(
        blk, blk, (((0,), (0,)), ((), ())), preferred_element_type=F32
    )


def _stats_body(E, npad, p_ref, sea_ref, crow_ref, ccol_ref, mea_ref, w1e_ref,
                b1_ref, g1_ref, be1_ref, pa_ref, w1es_ref, cnt_ref):
    n = p_ref.shape[0]
    P = p_ref[...]
    sea = sea_ref[0:n, :] + sea_ref[npad:npad + n, :]
    crow = crow_ref[0:n, 0:1] + crow_ref[npad:npad + n, 0:1]
    ccol = ccol_ref[0:n, 0:1] + ccol_ref[npad:npad + n, 0:1]
    w1e = w1e_ref[...]
    sp = jnp.sum(crow * P, axis=0, keepdims=True)
    sp2 = jnp.sum(crow * P * P, axis=0, keepdims=True)
    seaw = jnp.dot(sea, w1e, preferred_element_type=F32)
    spq = jnp.sum(P * seaw, axis=0, keepdims=True)
    s_ea = jnp.sum(sea, axis=0, keepdims=True)
    sq = jnp.dot(s_ea, w1e, preferred_element_type=F32)
    sq2 = jnp.sum(
        jnp.dot(mea_ref[...], w1e, preferred_element_type=F32) * w1e,
        axis=0, keepdims=True)
    b1 = b1_ref[...]
    mean = (sp + sq) / E + b1
    sh2 = sp2 + 2.0 * spq + sq2 + 2.0 * b1 * (sp + sq) + E * b1 * b1
    var = jnp.maximum(sh2 / E - mean * mean, 0.0)
    a = g1_ref[...] * lax.rsqrt(var + EPS)
    pa_ref[...] = a * P + (a * (b1 - mean) + be1_ref[...])
    w1es_ref[...] = w1e * a
    cnt_ref[...] = ccol


def _qa_body(ea_ref, w_ref, o_ref):
    o_ref[...] = jnp.dot(ea_ref[...], w_ref[...], preferred_element_type=F32)


def _e1_body(x_ref, b_ref, s0_ref, s1_ref, cnt_ref, w2x_ref, w2b_ref,
             w2a_ref, b2_ref, h2_ref, ssum_ref, ssq_ref):
    agg = (s0_ref[...] + s1_ref[...]) / jnp.maximum(cnt_ref[...], 1.0)
    h2 = (jnp.dot(x_ref[...], w2x_ref[...], preferred_element_type=F32)
          + jnp.dot(agg, w2a_ref[...], preferred_element_type=F32)
          + b_ref[...] * w2b_ref[...] + b2_ref[...])
    h2_ref[...] = h2

    @pl.when(pl.program_id(0) == 0)
    def _init():
        ssum_ref[...] = jnp.zeros_like(ssum_ref)
        ssq_ref[...] = jnp.zeros_like(ssq_ref)

    ssum_ref[...] += jnp.sum(h2, axis=0, keepdims=True)
    ssq_ref[...] += jnp.sum(h2 * h2, axis=0, keepdims=True)


def _e2_body(n_nodes, h2_ref, ssum_ref, ssq_ref, g2_ref, be2_ref, w3_ref,
             b3_ref, o_ref):
    m2 = ssum_ref[...] / n_nodes
    v2 = jnp.maximum(ssq_ref[...] / n_nodes - m2 * m2, 0.0)
    a2 = g2_ref[...] * lax.rsqrt(v2 + EPS)
    bb = be2_ref[...] - a2 * m2
    g = jnp.maximum(a2 * h2_ref[...] + bb, 0.0)
    o_ref[...] = jnp.maximum(
        jnp.dot(g, w3_ref[...], preferred_element_type=F32) + b3_ref[...],
        0.0)


def _sc_moments(row, col, ea, N):
    E, DE = ea.shape
    nchunk = E // CHUNK
    full, extra = nchunk // NW, nchunk % NW
    npad = _npad(N)
    rows_per = npad // 16
    mesh = plsc.VectorSubcoreMesh(core_axis_name="c", subcore_axis_name="s",
                                  num_cores=2, num_subcores=16)
    nblk = rows_per // CHUNK
    ones = jnp.ones((CHUNK, DE), F32)
    iota = jnp.arange(npad, dtype=jnp.int32)

    @functools.partial(
        pl.kernel, mesh=mesh,
        out_type=(_sds((2 * npad, DE)), _sds((2 * npad, DE)),
                  _sds((2 * npad, DE))),
        scratch_types=[
            pltpu.VMEM((CHUNK,), jnp.int32),
            pltpu.VMEM((CHUNK,), jnp.int32),
            pltpu.VMEM((CHUNK,), jnp.int32),
            pltpu.VMEM((CHUNK, DE), F32),
            pltpu.VMEM((CHUNK, DE), F32),
            pltpu.VMEM((CHUNK, DE), F32),
            pltpu.VMEM_SHARED((npad, DE), F32),
            pltpu.VMEM_SHARED((npad, DE), F32),
            pltpu.VMEM_SHARED((npad, DE), F32),
        ],
    )
    def k(row_hbm, col_hbm, ea_hbm, ones_hbm, iota_hbm,
          sea_out, crow_out, ccol_out,
          idxr_v, idxc_v, idxz_v, ea_v, ones_v, stage_v,
          sea_s, crow_s, ccol_s):
        c = lax.axis_index("c")
        s = lax.axis_index("s")
        w = s * 2 + c
        base = s * rows_per

        def zfill(i, carry):
            stage_v[i, pl.ds(0, 16)] = jnp.zeros((16,), F32)
            return carry

        lax.fori_loop(0, CHUNK, zfill, 0)
        pltpu.sync_copy(ones_hbm, ones_v)
        for b in range(nblk):
            pltpu.sync_copy(iota_hbm.at[pl.ds(base + b * CHUNK, CHUNK)],
                            idxz_v)
            pltpu.sync_copy(stage_v, sea_s.at[idxz_v])
            pltpu.sync_copy(stage_v, crow_s.at[idxz_v])
            pltpu.sync_copy(stage_v, ccol_s.at[idxz_v])
        plsc.subcore_barrier()

        nw = full + jnp.where(w < extra, 1, 0)

        def body(t, carry):
            g = w + NW * t
            off = pl.multiple_of(g * CHUNK, CHUNK)
            pltpu.sync_copy(row_hbm.at[pl.ds(off, CHUNK)], idxr_v)
            pltpu.sync_copy(col_hbm.at[pl.ds(off, CHUNK)], idxc_v)
            pltpu.sync_copy(ea_hbm.at[pl.ds(off, CHUNK)], ea_v)
            pltpu.sync_copy(ea_v, sea_s.at[idxr_v], add=True)
            pltpu.sync_copy(ones_v, crow_s.at[idxr_v], add=True)
            pltpu.sync_copy(ones_v, ccol_s.at[idxc_v], add=True)
            return carry

        lax.fori_loop(0, nw, body, 0)
        plsc.subcore_barrier()

        obase = c * npad + base
        for b in range(nblk):
            pltpu.sync_copy(iota_hbm.at[pl.ds(base + b * CHUNK, CHUNK)],
                            idxz_v)
            oslc = pl.ds(obase + b * CHUNK, CHUNK)
            pltpu.sync_copy(sea_s.at[idxz_v], stage_v)
            pltpu.sync_copy(stage_v, sea_out.at[oslc])
            pltpu.sync_copy(crow_s.at[idxz_v], stage_v)
            pltpu.sync_copy(stage_v, crow_out.at[oslc])
            pltpu.sync_copy(ccol_s.at[idxz_v], stage_v)
            pltpu.sync_copy(stage_v, ccol_out.at[oslc])

    return k(row, col, ea, ones, iota)


def _sc_edge_pass(row, col, pa, qa, N):
    E = row.shape[0]
    D = pa.shape[1]
    nchunk = E // CHUNK
    full, extra = nchunk // NW, nchunk % NW
    npad = _npad(N)
    rows_per = npad // 16
    mesh = plsc.VectorSubcoreMesh(core_axis_name="c", subcore_axis_name="s",
                                  num_cores=2, num_subcores=16)
    nblk = rows_per // CHUNK

    @functools.partial(
        pl.kernel, mesh=mesh,
        out_type=_sds((2 * npad, D)),
        scratch_types=[
            pltpu.VMEM((CHUNK,), jnp.int32),
            pltpu.VMEM((CHUNK,), jnp.int32),
            pltpu.VMEM((CHUNK, D), F32),
            pltpu.VMEM((CHUNK, D), F32),
            pltpu.VMEM_SHARED((npad, D), F32),
            pltpu.SemaphoreType.DMA,
        ],
    )
    def k(row_hbm, col_hbm, pa_hbm, qa_hbm, sums_out,
          idxr_v, idxc_v, pg_v, qa_v, sums_s, sem):
        c = lax.axis_index("c")
        s = lax.axis_index("s")
        w = s * 2 + c
        base = s * rows_per

        def zfill(i, carry):
            for j in range(D // 16):
                pg_v[i, pl.ds(j * 16, 16)] = jnp.zeros((16,), F32)
            return carry

        lax.fori_loop(0, CHUNK, zfill, 0)
        for b in range(nblk):
            pltpu.sync_copy(pg_v, sums_s.at[pl.ds(base + b * CHUNK, CHUNK)])
        plsc.subcore_barrier()

        nw = full + jnp.where(w < extra, 1, 0)

        def body(t, carry):
            g = w + NW * t
            off = pl.multiple_of(g * CHUNK, CHUNK)
            pltpu.sync_copy(row_hbm.at[pl.ds(off, CHUNK)], idxr_v)
            pltpu.sync_copy(col_hbm.at[pl.ds(off, CHUNK)], idxc_v)
            gcp = pltpu.async_copy(pa_hbm.at[idxr_v], pg_v, sem)
            pltpu.sync_copy(qa_hbm.at[pl.ds(off, CHUNK)], qa_v)
            gcp.wait()

            def edge(i, cc):
                for j in range(D // 16):
                    sl = pl.ds(j * 16, 16)
                    pg_v[i, sl] = jnp.maximum(pg_v[i, sl] + qa_v[i, sl], 0.0)
                return cc

            lax.fori_loop(0, CHUNK, edge, 0)
            pltpu.sync_copy(pg_v, sums_s.at[idxc_v], add=True)
            return carry

        lax.fori_loop(0, nw, body, 0)
        plsc.subcore_barrier()
        obase = c * npad + base
        for b in range(nblk):
            pltpu.sync_copy(sums_s.at[pl.ds(base + b * CHUNK, CHUNK)], pg_v)
            pltpu.sync_copy(pg_v, sums_out.at[pl.ds(obase + b * CHUNK, CHUNK)])

    return k(row, col, pa, qa)


def kernel(x, edge_index, edge_attr, u, batch, W1, b1, g1, be1,
           W2, b2, g2, be2, W3, b3):
    del u
    N, D = x.shape
    E, DE = edge_attr.shape
    L = W1.shape[1]
    d2 = W2.shape[0]
    assert E % CHUNK == 0 and N % 16 == 0 and D % 16 == 0

    row = edge_index[0]
    col = edge_index[1]
    bcol = batch[:, None]
    w1x, w1b, w1e = W1[:D], W1[D:D + 1], W1[D + 1:]
    w2x, w2b, w2a = W2[:D], W2[D:D + 1], W2[D + 1:]

    P = pl.pallas_call(
        _p_body, out_shape=_sds((N, D)),
    )(x, bcol, w1x, w1b)

    bm = 8000
    mea = pl.pallas_call(
        _mea_body,
        grid=(E // bm,),
        in_specs=[pl.BlockSpec((bm, DE), lambda i: (i, 0))],
        out_specs=pl.BlockSpec((DE, DE), lambda i: (0, 0)),
        out_shape=_sds((DE, DE)),
    )(edge_attr)

    npad0 = _npad(N)
    sea_p = jnp.zeros((2 * npad0, DE), F32).at[:N].add(
        jnp.zeros((N, DE), F32).at[row].add(edge_attr))
    crow_p = jnp.zeros((2 * npad0, DE), F32).at[:N, 0].add(
        jnp.zeros((N,), F32).at[row].add(1.0))
    ccol_p = jnp.zeros((2 * npad0, DE), F32).at[:N, 0].add(
        jnp.zeros((N,), F32).at[col].add(1.0))

    PA, w1e_s, cnt = pl.pallas_call(
        functools.partial(_stats_body, float(E), _npad(N)),
        out_shape=(_sds((N, D)), _sds((DE, L)), _sds((N, 1))),
    )(P, sea_p, crow_p, ccol_p, mea, w1e,
      b1[None, :], g1[None, :], be1[None, :])

    bq = 8000
    qa = pl.pallas_call(
        _qa_body,
        grid=(E // bq,),
        in_specs=[pl.BlockSpec((bq, DE), lambda i: (i, 0)),
                  pl.BlockSpec((DE, L), lambda i: (0, 0))],
        out_specs=pl.BlockSpec((bq, L), lambda i: (i, 0)),
        out_shape=_sds((E, L)),
    )(edge_attr, w1e_s)

    sums_p = _sc_edge_pass(row, col, PA, qa, N)
    s0 = sums_p[:npad0][:N]
    s1 = sums_p[npad0:npad0 + N]

    bn = 2000
    nb = N // bn
    h2, ssum, ssq = pl.pallas_call(
        _e1_body,
        grid=(nb,),
        in_specs=[
            pl.BlockSpec((bn, D), lambda i: (i, 0)),
            pl.BlockSpec((bn, 1), lambda i: (i, 0)),
            pl.BlockSpec((bn, L), lambda i: (i, 0)),
            pl.BlockSpec((bn, L), lambda i: (i, 0)),
            pl.BlockSpec((bn, 1), lambda i: (i, 0)),
            pl.BlockSpec((D, d2), lambda i: (0, 0)),
            pl.BlockSpec((1, d2), lambda i: (0, 0)),
            pl.BlockSpec((L, d2), lambda i: (0, 0)),
            pl.BlockSpec((1, d2), lambda i: (0, 0)),
        ],
        out_specs=(
            pl.BlockSpec((bn, d2), lambda i: (i, 0)),
            pl.BlockSpec((1, d2), lambda i: (0, 0)),
            pl.BlockSpec((1, d2), lambda i: (0, 0)),
        ),
        out_shape=(_sds((N, d2)), _sds((1, d2)), _sds((1, d2))),
    )(x, bcol, s0, s1, cnt, w2x, w2b, w2a, b2[None, :])

    out = pl.pallas_call(
        functools.partial(_e2_body, float(N)),
        grid=(nb,),
        in_specs=[
            pl.BlockSpec((bn, d2), lambda i: (i, 0)),
            pl.BlockSpec((1, d2), lambda i: (0, 0)),
            pl.BlockSpec((1, d2), lambda i: (0, 0)),
            pl.BlockSpec((1, d2), lambda i: (0, 0)),
            pl.BlockSpec((1, d2), lambda i: (0, 0)),
            pl.BlockSpec((d2, D), lambda i: (0, 0)),
            pl.BlockSpec((1, D), lambda i: (0, 0)),
        ],
        out_specs=pl.BlockSpec((bn, D), lambda i: (i, 0)),
        out_shape=_sds((N, D)),
    )(h2, ssum, ssq, g2[None, :], be2[None, :], W3, b3[None, :])

    return out
